# Initial kernel scaffold; baseline (speedup 1.0000x reference)
import jax
import jax.numpy as jnp
from jax.experimental import pallas as pl


def kernel(x, environments, Wi1, bi1, Wi2, bi2, Ws1, bs1, Ws2, bs2, Wp, bp, Wd1, bd1, Wd2, bd2):
    raise NotImplementedError("write your pallas kernel here")


# speedup vs baseline: 1.0870x; 1.0870x over previous
import functools

import jax
import jax.numpy as jnp
from jax.experimental import pallas as pl
from jax.experimental.pallas import tpu as pltpu

B, D, E = 8192, 1024, 8
H, INV, SPEC = 128, 64, 32
H2 = H // 2
TILE = 512


def _fused_body(env_ref, x_ref, Wi1_ref, bi1_ref, Wi2_ref, bi2_ref,
                Ws1_ref, bs1_ref, Ws2_ref, bs2_ref, Wp_ref, bp_ref,
                Wd1_ref, bd1_ref, Wd2_ref, bd2_ref,
                logits_ref, inv_ref, spec_ref, dl_ref):
    xb = x_ref[...]
    f32 = jnp.float32
    h = jnp.maximum(jnp.dot(xb, Wi1_ref[...], preferred_element_type=f32)
                    + bi1_ref[...], 0.0)
    inv = jnp.dot(h, Wi2_ref[...], preferred_element_type=f32) + bi2_ref[...]
    inv_ref[...] = inv
    logits_ref[...] = jnp.dot(inv, Wp_ref[...], preferred_element_type=f32) + bp_ref[...]
    dh = jnp.maximum(jnp.dot(inv, Wd1_ref[...], preferred_element_type=f32)
                     + bd1_ref[...], 0.0)
    dl_ref[...] = jnp.dot(dh, Wd2_ref[...], preferred_element_type=f32) + bd2_ref[...]

    env = env_ref[...]
    acc = jnp.zeros((TILE, SPEC), dtype=f32)
    for e in range(E):
        he = jnp.maximum(jnp.dot(xb, Ws1_ref[e], preferred_element_type=f32)
                         + bs1_ref[e][None, :], 0.0)
        se = jnp.dot(he, Ws2_ref[e], preferred_element_type=f32) + bs2_ref[e][None, :]
        acc = acc + jnp.where(env == e, se, 0.0)
    spec_ref[...] = acc


def kernel(x, environments, Wi1, bi1, Wi2, bi2, Ws1, bs1, Ws2, bs2,
           Wp, bp, Wd1, bd1, Wd2, bd2):
    env2 = environments.reshape(B, 1)
    grid = (B // TILE,)
    row_spec = lambda w: pl.BlockSpec((TILE, w), lambda i: (i, 0))
    full = lambda a: pl.BlockSpec(a.shape, lambda i: (0,) * a.ndim)
    out = pl.pallas_call(
        _fused_body,
        grid=grid,
        in_specs=[
            row_spec(1),
            row_spec(D),
            full(Wi1), full(bi1.reshape(1, H)),
            full(Wi2), full(bi2.reshape(1, INV)),
            full(Ws1), full(bs1),
            full(Ws2), full(bs2),
            full(Wp), full(bp.reshape(1, 1)),
            full(Wd1), full(bd1.reshape(1, H2)),
            full(Wd2), full(bd2.reshape(1, E)),
        ],
        out_specs=[row_spec(1), row_spec(INV), row_spec(SPEC), row_spec(E)],
        out_shape=[
            jax.ShapeDtypeStruct((B, 1), jnp.float32),
            jax.ShapeDtypeStruct((B, INV), jnp.float32),
            jax.ShapeDtypeStruct((B, SPEC), jnp.float32),
            jax.ShapeDtypeStruct((B, E), jnp.float32),
        ],
    )(env2, x, Wi1, bi1.reshape(1, H), Wi2, bi2.reshape(1, INV),
      Ws1, bs1, Ws2, bs2, Wp, bp.reshape(1, 1),
      Wd1, bd1.reshape(1, H2), Wd2, bd2.reshape(1, E))
    logits, invariant, specific, domain_logits = out
    return (logits, invariant, specific, domain_logits)
